# baseline (speedup 1.0000x reference)
import functools

import jax
import jax.numpy as jnp
from jax import lax
from jax.experimental import pallas as pl
from jax.experimental.pallas import tpu as pltpu
from jax.experimental.pallas import tpu_sc as plsc

_S = 10000
_EX = 10000
_K = 128
_D = 32
_B = 16384

_INFO = plsc.get_sparse_core_info()
_NW = _INFO.num_cores * _INFO.num_subcores
_BPW = _B // _NW
_EDW = 16


def _sc_gather(stu_id_h, ex_id_h, stu_tab_h, ex_tab_h, ed_tab_h,
               out_s_h, out_e_h, out_d_h,
               sidx, eidx, srows, erows, drows, sem):
    wid = lax.axis_index("s") * _INFO.num_cores + lax.axis_index("c")
    base = wid * _BPW
    pltpu.sync_copy(stu_id_h.at[pl.ds(base, _BPW)], sidx)
    pltpu.sync_copy(ex_id_h.at[pl.ds(base, _BPW)], eidx)
    c1 = pltpu.async_copy(stu_tab_h.at[sidx], srows, sem)
    c2 = pltpu.async_copy(ex_tab_h.at[eidx], erows, sem)
    c3 = pltpu.async_copy(ed_tab_h.at[eidx], drows, sem)
    c1.wait()
    c2.wait()
    c3.wait()
    pltpu.sync_copy(srows, out_s_h.at[pl.ds(base, _BPW)])
    pltpu.sync_copy(erows, out_e_h.at[pl.ds(base, _BPW)])
    pltpu.sync_copy(drows, out_d_h.at[pl.ds(base, _BPW)])


_sc_gather_call = functools.partial(
    pl.kernel,
    mesh=plsc.VectorSubcoreMesh(core_axis_name="c", subcore_axis_name="s"),
    compiler_params=pltpu.CompilerParams(use_tc_tiling_on_sc=False),
    out_type=[
        jax.ShapeDtypeStruct((_B, _D), jnp.float32),
        jax.ShapeDtypeStruct((_B, _D), jnp.float32),
        jax.ShapeDtypeStruct((_B, _EDW), jnp.float32),
    ],
    scratch_types=[
        pltpu.VMEM((_BPW,), jnp.int32),
        pltpu.VMEM((_BPW,), jnp.int32),
        pltpu.VMEM((_BPW, _D), jnp.float32),
        pltpu.VMEM((_BPW, _D), jnp.float32),
        pltpu.VMEM((_BPW, _EDW), jnp.float32),
        pltpu.SemaphoreType.DMA,
    ],
)(_sc_gather)


def _tc_dense(ikp_ref, gs_ref, ge_ref, ed_ref, kemb_ref, out_ref):
    dn = (((1,), (1,)), ((), ()))
    kemb = kemb_ref[...]
    se = jax.nn.sigmoid(lax.dot_general(gs_ref[...], kemb, dn,
                                        preferred_element_type=jnp.float32))
    kd = lax.dot_general(ge_ref[...], kemb, dn,
                         preferred_element_type=jnp.float32)
    ikp = ikp_ref[...]
    num = jnp.sum(ikp * (se - kd), axis=1, keepdims=True)
    den = jnp.sum(ikp, axis=1, keepdims=True)
    ed = jax.nn.sigmoid(ed_ref[:, 0:1])
    out_ref[...] = jax.nn.sigmoid(ed * num / den)


def kernel(stu_id, input_exercise, input_knowledge_point, student_emb,
           exercise_emb, knowledge_emb, e_disc, edge_index_1, edge_vals_1,
           edge_index_0, edge_vals_0, d_i_1, d_j_1, d_i_0, d_j_0):
    ed_tab = jnp.pad(e_disc, ((0, 0), (0, _EDW - 1)))
    gs, ge, ed = _sc_gather_call(
        stu_id.astype(jnp.int32), input_exercise.astype(jnp.int32),
        student_emb, exercise_emb, ed_tab)

    bb = 2048
    grid = _B // bb
    out = pl.pallas_call(
        _tc_dense,
        grid=(grid,),
        in_specs=[
            pl.BlockSpec((bb, _K), lambda i: (i, 0)),
            pl.BlockSpec((bb, _D), lambda i: (i, 0)),
            pl.BlockSpec((bb, _D), lambda i: (i, 0)),
            pl.BlockSpec((bb, _EDW), lambda i: (i, 0)),
            pl.BlockSpec((_K, _D), lambda i: (0, 0)),
        ],
        out_specs=pl.BlockSpec((bb, 1), lambda i: (i, 0)),
        out_shape=jax.ShapeDtypeStruct((_B, 1), jnp.float32),
    )(input_knowledge_point, gs, ge, ed, knowledge_emb)
    return out.reshape(-1)
